# baseline (speedup 1.0000x reference)
import functools

import jax
import jax.numpy as jnp
from jax import lax
from jax.experimental import pallas as pl
from jax.experimental.pallas import tpu as pltpu
from jax.experimental.pallas import tpu_sc as plsc

U = 10000
D = 256
DH = 128
E = 160000
NC = 2
NS = 16
LANES = 16
K = 64
E_PAD = 163840
ROWS_B = E_PAD // K
CHUNKS = E_PAD // NS // K
SCHUNKS = 16
NSTAGE = CHUNKS // SCHUNKS
DRAIN_TILES = 10
DRAIN_ROWS = U // DRAIN_TILES
ZROWS = 8
IDX128 = [g * 32 + (p // 2 + 16 * (p % 2)) for g in range(4) for p in range(32)]
BM = 1000


def _sc_layer_body(dst_u, dst_i, vals, ie_cat, ue_cat, out_u, out_i,
                   sidx_v, didx_v, vals_v, g0_v, g1_v, s0_v, s1_v, zero_v,
                   acc_sh, gsem0, gsem1, ssem0, ssem1, zsem):
    cid = lax.axis_index("c")
    sid = lax.axis_index("s")
    drow0 = sid * DRAIN_ROWS
    half_off = jnp.full((LANES,), cid * U, jnp.int32)

    zvec = jnp.zeros((LANES,), jnp.float32)

    @plsc.parallel_loop(0, ZROWS, 1, unroll=2)
    def _(i):
        for t in range(DH // LANES):
            zero_v[i, pl.ds(t * LANES, LANES)] = zvec

    gbufs = (g0_v, g1_v)
    sbufs = (s0_v, s1_v)
    gsems = (gsem0, gsem1)
    ssems = (ssem0, ssem1)

    def gather_start(x_cat, jj, b):
        pltpu.async_copy(x_cat.at[sidx_v.at[jj]], gbufs[b], gsems[b])

    def gather_wait(x_cat, jj, b):
        pltpu.make_async_copy(
            x_cat.at[sidx_v.at[jj]], gbufs[b], gsems[b]).wait()

    def scatter_start(jj, b):
        pltpu.async_copy(sbufs[b], acc_sh.at[didx_v.at[jj]], ssems[b],
                         add=True)

    def scatter_wait(b):
        pltpu.make_async_copy(
            sbufs[b], acc_sh.at[pl.ds(0, K)], ssems[b]).wait()

    himask = jnp.full((LANES,), -65536, jnp.int32)

    def scale(jj, b):
        gbuf, sbuf = gbufs[b], sbufs[b]

        @plsc.parallel_loop(0, K, 1, unroll=2)
        def _(k):
            vk = plsc.load_gather(
                vals_v,
                [jnp.full((LANES,), jj, jnp.int32),
                 jnp.full((LANES,), k, jnp.int32)])
            for g in range(DH // 32):
                w = gbuf[k, pl.ds(g * LANES, LANES)]
                lo = plsc.bitcast(w << 16, jnp.float32)
                hi = plsc.bitcast(w & himask, jnp.float32)
                sbuf[k, pl.ds(g * 32, LANES)] = lo * vk
                sbuf[k, pl.ds(g * 32 + LANES, LANES)] = hi * vk

    def stage_pass(dst_rc, src_rc, crow0):
        pltpu.sync_copy(src_rc.at[pl.ds(crow0, SCHUNKS)], sidx_v)
        pltpu.sync_copy(dst_rc.at[pl.ds(crow0, SCHUNKS)], didx_v)
        pltpu.sync_copy(vals.at[pl.ds(crow0, SCHUNKS)], vals_v)

        @plsc.parallel_loop(0, SCHUNKS, 1, unroll=2)
        def _(r):
            for t in range(K // LANES):
                sidx_v[r, pl.ds(t * LANES, LANES)] = (
                    sidx_v[r, pl.ds(t * LANES, LANES)] + half_off)

    def one_direction(dst_rc, src_rc, x_cat, out_ref):
        @pl.when(sid < DRAIN_TILES)
        def _():
            def zb(z, c):
                pltpu.async_copy(
                    zero_v, acc_sh.at[pl.ds(drow0 + z * ZROWS, ZROWS)], zsem)
                return c

            lax.fori_loop(0, DRAIN_ROWS // ZROWS, zb, 0)

            def zw(z, c):
                pltpu.make_async_copy(
                    zero_v, acc_sh.at[pl.ds(drow0, ZROWS)], zsem).wait()
                return c

            lax.fori_loop(0, DRAIN_ROWS // ZROWS, zw, 0)

        plsc.subcore_barrier()

        def behavior(b, carry):
            def stage(st, c2):
                crow0 = b * ROWS_B + sid * CHUNKS + st * SCHUNKS
                stage_pass(dst_rc, src_rc, crow0)

                gather_start(x_cat, 0, 0)

                def pair(j2, c3):
                    j0 = j2 * 2
                    gather_wait(x_cat, j0, 0)

                    @pl.when(j2 >= 1)
                    def _():
                        scatter_wait(1)

                    gather_start(x_cat, j0 + 1, 1)
                    scale(j0, 0)
                    scatter_start(j0, 0)
                    gather_wait(x_cat, j0 + 1, 1)

                    @pl.when(j2 + 1 < SCHUNKS // 2)
                    def _():
                        scatter_wait(0)
                        gather_start(x_cat, j0 + 2, 0)

                    scale(j0 + 1, 1)
                    scatter_start(j0 + 1, 1)
                    return c3

                lax.fori_loop(0, SCHUNKS // 2, pair, 0)
                scatter_wait(0)
                scatter_wait(1)
                return c2

            lax.fori_loop(0, NSTAGE, stage, 0)
            plsc.subcore_barrier()

            @pl.when(sid < DRAIN_TILES)
            def _():
                pltpu.sync_copy(
                    acc_sh.at[pl.ds(drow0, DRAIN_ROWS)],
                    out_ref.at[b, pl.ds(drow0, DRAIN_ROWS),
                               pl.ds(cid * DH, DH)])
            plsc.subcore_barrier()
            return carry

        lax.fori_loop(0, 3, behavior, 0)

    one_direction(dst_u, dst_i, ie_cat, out_u)
    one_direction(dst_i, dst_u, ue_cat, out_i)


def _sc_layer(dst_u, dst_i, vals, ie_cat, ue_cat):
    mesh = plsc.VectorSubcoreMesh(
        core_axis_name="c", subcore_axis_name="s",
        num_cores=NC, num_subcores=NS)
    f = pl.kernel(
        _sc_layer_body,
        out_type=[jax.ShapeDtypeStruct((3, U, D), jnp.float32),
                  jax.ShapeDtypeStruct((3, U, D), jnp.float32)],
        mesh=mesh,
        scratch_types=[
            pltpu.VMEM((SCHUNKS, K), jnp.int32),
            pltpu.VMEM((SCHUNKS, K), jnp.int32),
            pltpu.VMEM((SCHUNKS, K), jnp.float32),
            pltpu.VMEM((K, DH // 2), jnp.int32),
            pltpu.VMEM((K, DH // 2), jnp.int32),
            pltpu.VMEM((K, DH), jnp.float32),
            pltpu.VMEM((K, DH), jnp.float32),
            pltpu.VMEM((ZROWS, DH), jnp.float32),
            pltpu.VMEM_SHARED((U, DH), jnp.float32),
            pltpu.SemaphoreType.DMA,
            pltpu.SemaphoreType.DMA,
            pltpu.SemaphoreType.DMA,
            pltpu.SemaphoreType.DMA,
            pltpu.SemaphoreType.DMA,
        ],
        compiler_params=pltpu.CompilerParams(needs_layout_passes=False,
                                             use_tc_tiling_on_sc=False),
    )
    return f(dst_u, dst_i, vals, ie_cat, ue_cat)


def _tc_layer_kernel(a_ref, w_ref, wp_ref, al_ref, s_ref, m_ref, mp_ref):
    al = al_ref[0, 0]
    w = w_ref[...]
    dot = functools.partial(jnp.dot, preferred_element_type=jnp.float32)
    z_prev = None
    for b in range(3):
        z = dot(a_ref[b], w)
        y = z if z_prev is None else z - z_prev
        s_ref[b, :, :] = jnp.where(y > 0, y, al * y)
        z_prev = z
    m = z_prev * (1.0 / 3.0)
    m = jnp.where(m > 0, m, al * m)
    m_ref[0, :, :] = m[:, :DH]
    m_ref[1, :, :] = m[:, DH:]
    zp = dot(a_ref[2], wp_ref[...]) * (1.0 / 3.0)
    mp = jnp.where(zp > 0, zp, al * zp).astype(jnp.bfloat16)
    mp_ref[0, :, :] = mp[:, :DH]
    mp_ref[1, :, :] = mp[:, DH:]


def _tc_layer(embs, w, wp, alpha):
    return pl.pallas_call(
        _tc_layer_kernel,
        grid=(U // BM,),
        in_specs=[
            pl.BlockSpec((3, BM, D), lambda i: (0, i, 0)),
            pl.BlockSpec((D, D), lambda i: (0, 0)),
            pl.BlockSpec((D, D), lambda i: (0, 0)),
            pl.BlockSpec(memory_space=pltpu.SMEM),
        ],
        out_specs=[
            pl.BlockSpec((3, BM, D), lambda i: (0, i, 0)),
            pl.BlockSpec((2, BM, DH), lambda i: (0, i, 0)),
            pl.BlockSpec((2, BM, DH), lambda i: (0, i, 0)),
        ],
        out_shape=[
            jax.ShapeDtypeStruct((3, U, D), jnp.float32),
            jax.ShapeDtypeStruct((2, U, DH), jnp.float32),
            jax.ShapeDtypeStruct((2, U, DH), jnp.bfloat16),
        ],
    )(embs, w, wp, alpha.reshape(1, 1))


def _tc_final_kernel(m0_ref, m1_ref, s0_ref, s1_ref, w_ref, emb_ref, embs_ref):
    w = w_ref[...]
    dot = functools.partial(jnp.dot, preferred_element_type=jnp.float32)
    emb_ref[...] = (dot(m0_ref[0], w[:DH])
                    + dot(m0_ref[1], w[DH:D])
                    + dot(m1_ref[0], w[D:D + DH])
                    + dot(m1_ref[1], w[D + DH:]))
    for b in range(3):
        embs_ref[b, :, :] = (dot(s0_ref[b], w[:D]) + dot(s1_ref[b], w[D:]))


def _tc_final(m0, m1, s0, s1, wcat):
    hspec = pl.BlockSpec((2, BM, DH), lambda i: (0, i, 0))
    sspec = pl.BlockSpec((3, BM, D), lambda i: (0, i, 0))
    return pl.pallas_call(
        _tc_final_kernel,
        grid=(U // BM,),
        in_specs=[
            hspec, hspec,
            sspec, sspec,
            pl.BlockSpec((2 * D, D), lambda i: (0, 0)),
        ],
        out_specs=[
            pl.BlockSpec((BM, D), lambda i: (i, 0)),
            sspec,
        ],
        out_shape=[
            jax.ShapeDtypeStruct((U, D), jnp.float32),
            jax.ShapeDtypeStruct((3, U, D), jnp.float32),
        ],
    )(m0, m1, s0, s1, wcat)


def kernel(user_emb, item_emb, rows0, cols0, vals0, rows1, cols1, vals1,
           rows2, cols2, vals2, u_w0, i_w0, alpha0, u_w1, i_w1, alpha1,
           u_concat_w, i_concat_w):
    pad_idx = jnp.arange(E_PAD - E, dtype=jnp.int32)
    pad_val = jnp.zeros((E_PAD - E,), jnp.float32)

    def prep_i(x):
        return jnp.concatenate([x.astype(jnp.int32), pad_idx]).reshape(
            ROWS_B, K)

    def prep_f(x):
        return jnp.concatenate([x, pad_val]).reshape(ROWS_B, K)

    dst_u = jnp.concatenate([prep_i(rows0), prep_i(rows1), prep_i(rows2)])
    dst_i = jnp.concatenate([prep_i(cols0), prep_i(cols1), prep_i(cols2)])
    vals = jnp.concatenate([prep_f(vals0), prep_f(vals1), prep_f(vals2)])

    perm256 = jnp.array(IDX128 + [DH + j for j in IDX128], dtype=jnp.int32)

    def pack_i32(xb):
        return jax.lax.bitcast_convert_type(
            xb.reshape(2 * U, DH // 2, 2), jnp.int32)

    def to_cat(x):
        xp = x[:, perm256].astype(jnp.bfloat16)
        xc = jnp.transpose(xp.reshape(U, 2, DH), (1, 0, 2)).reshape(2 * U, DH)
        return pack_i32(xc)

    ue_cat = to_cat(user_emb)
    ie_cat = to_cat(item_emb)

    u_embs0, i_embs0 = _sc_layer(dst_u, dst_i, vals, ie_cat, ue_cat)
    wp_u0, wp_i0 = u_w0[:, perm256], i_w0[:, perm256]
    s_u0, mu0, mpu0 = _tc_layer(u_embs0, u_w0, wp_u0, alpha0)
    s_i0, mi0, mpi0 = _tc_layer(i_embs0, i_w0, wp_i0, alpha0)

    u_embs1, i_embs1 = _sc_layer(dst_u, dst_i, vals,
                                 pack_i32(mpi0.reshape(2 * U, DH)),
                                 pack_i32(mpu0.reshape(2 * U, DH)))
    wp_u1, wp_i1 = u_w1[:, perm256], i_w1[:, perm256]
    s_u1, mu1, _ = _tc_layer(u_embs1, u_w1, wp_u1, alpha1)
    s_i1, mi1, _ = _tc_layer(i_embs1, i_w1, wp_i1, alpha1)

    user_embedding, user_embeddings = _tc_final(mu0, mu1, s_u0, s_u1,
                                                u_concat_w)
    item_embedding, item_embeddings = _tc_final(mi0, mi1, s_i0, s_i1,
                                                i_concat_w)

    return (user_embedding, item_embedding, user_embeddings, item_embeddings)


# speedup vs baseline: 1.1525x; 1.1525x over previous
import functools

import jax
import jax.numpy as jnp
from jax import lax
from jax.experimental import pallas as pl
from jax.experimental.pallas import tpu as pltpu
from jax.experimental.pallas import tpu_sc as plsc

U = 10000
D = 256
DH = 128
E = 160000
NC = 2
NS = 16
LANES = 16
K = 80
E_PAD = 163840
ROWS_B = E_PAD // K
CHUNKS = E_PAD // NS // K
SCHUNKS = 16
NSTAGE = CHUNKS // SCHUNKS
DRAIN_TILES = 10
DRAIN_ROWS = U // DRAIN_TILES
ZROWS = 40
BM = 1000


def _sc_layer_body(dst_u, dst_i, vals, ie_cat, ue_cat, out_u, out_i,
                   sidx_v, didx_v, vals_v, rows0_v, rows1_v, zero_v, acc_sh,
                   gsem0, gsem1, ssem0, ssem1, zsem):
    cid = lax.axis_index("c")
    sid = lax.axis_index("s")
    drow0 = sid * DRAIN_ROWS
    half_off = jnp.full((LANES,), cid * U, jnp.int32)

    zvec = jnp.zeros((LANES,), jnp.float32)

    @plsc.parallel_loop(0, ZROWS, 1, unroll=2)
    def _(i):
        for t in range(DH // LANES):
            zero_v[i, pl.ds(t * LANES, LANES)] = zvec

    rows_bufs = (rows0_v, rows1_v)
    gsems = (gsem0, gsem1)
    ssems = (ssem0, ssem1)

    def gather_start(x_cat, jj, b):
        pltpu.async_copy(x_cat.at[sidx_v.at[jj]], rows_bufs[b], gsems[b])

    def gather_wait(x_cat, jj, b):
        pltpu.make_async_copy(
            x_cat.at[sidx_v.at[jj]], rows_bufs[b], gsems[b]).wait()

    def scatter_start(jj, b):
        pltpu.async_copy(rows_bufs[b], acc_sh.at[didx_v.at[jj]], ssems[b],
                         add=True)

    def scatter_wait(b):
        pltpu.make_async_copy(
            rows_bufs[b], acc_sh.at[pl.ds(0, K)], ssems[b]).wait()

    def scale(jj, b):
        buf = rows_bufs[b]

        @plsc.parallel_loop(0, K, 1, unroll=2)
        def _(k):
            vk = plsc.load_gather(
                vals_v,
                [jnp.full((LANES,), jj, jnp.int32),
                 jnp.full((LANES,), k, jnp.int32)])
            for t in range(DH // LANES):
                buf[k, pl.ds(t * LANES, LANES)] = (
                    buf[k, pl.ds(t * LANES, LANES)] * vk)

    def stage_pass(dst_rc, src_rc, crow0):
        pltpu.sync_copy(src_rc.at[pl.ds(crow0, SCHUNKS)], sidx_v)
        pltpu.sync_copy(dst_rc.at[pl.ds(crow0, SCHUNKS)], didx_v)
        pltpu.sync_copy(vals.at[pl.ds(crow0, SCHUNKS)], vals_v)

        @plsc.parallel_loop(0, SCHUNKS, 1, unroll=2)
        def _(r):
            for t in range(K // LANES):
                sidx_v[r, pl.ds(t * LANES, LANES)] = (
                    sidx_v[r, pl.ds(t * LANES, LANES)] + half_off)

    def one_direction(dst_rc, src_rc, x_cat, out_ref):
        @pl.when(sid < DRAIN_TILES)
        def _():
            def zb(z, c):
                pltpu.async_copy(
                    zero_v, acc_sh.at[pl.ds(drow0 + z * ZROWS, ZROWS)], zsem)
                return c

            lax.fori_loop(0, DRAIN_ROWS // ZROWS, zb, 0)

            def zw(z, c):
                pltpu.make_async_copy(
                    zero_v, acc_sh.at[pl.ds(drow0, ZROWS)], zsem).wait()
                return c

            lax.fori_loop(0, DRAIN_ROWS // ZROWS, zw, 0)

        plsc.subcore_barrier()

        def behavior(b, carry):
            def stage(st, c2):
                crow0 = b * ROWS_B + sid * CHUNKS + st * SCHUNKS
                stage_pass(dst_rc, src_rc, crow0)

                gather_start(x_cat, 0, 0)

                def pair(j2, c3):
                    j0 = j2 * 2
                    gather_wait(x_cat, j0, 0)

                    @pl.when(j2 >= 1)
                    def _():
                        scatter_wait(1)

                    gather_start(x_cat, j0 + 1, 1)
                    scale(j0, 0)
                    scatter_start(j0, 0)
                    gather_wait(x_cat, j0 + 1, 1)

                    @pl.when(j2 + 1 < SCHUNKS // 2)
                    def _():
                        scatter_wait(0)
                        gather_start(x_cat, j0 + 2, 0)

                    scale(j0 + 1, 1)
                    scatter_start(j0 + 1, 1)
                    return c3

                lax.fori_loop(0, SCHUNKS // 2, pair, 0)
                scatter_wait(0)
                scatter_wait(1)
                return c2

            lax.fori_loop(0, NSTAGE, stage, 0)
            plsc.subcore_barrier()

            @pl.when(sid < DRAIN_TILES)
            def _():
                pltpu.sync_copy(
                    acc_sh.at[pl.ds(drow0, DRAIN_ROWS)],
                    out_ref.at[b, pl.ds(drow0, DRAIN_ROWS),
                               pl.ds(cid * DH, DH)])
            plsc.subcore_barrier()
            return carry

        lax.fori_loop(0, 3, behavior, 0)

    one_direction(dst_u, dst_i, ie_cat, out_u)
    one_direction(dst_i, dst_u, ue_cat, out_i)


def _sc_layer(dst_u, dst_i, vals, ie_cat, ue_cat):
    mesh = plsc.VectorSubcoreMesh(
        core_axis_name="c", subcore_axis_name="s",
        num_cores=NC, num_subcores=NS)
    f = pl.kernel(
        _sc_layer_body,
        out_type=[jax.ShapeDtypeStruct((3, U, D), jnp.float32),
                  jax.ShapeDtypeStruct((3, U, D), jnp.float32)],
        mesh=mesh,
        scratch_types=[
            pltpu.VMEM((SCHUNKS, K), jnp.int32),
            pltpu.VMEM((SCHUNKS, K), jnp.int32),
            pltpu.VMEM((SCHUNKS, K), jnp.float32),
            pltpu.VMEM((K, DH), jnp.float32),
            pltpu.VMEM((K, DH), jnp.float32),
            pltpu.VMEM((ZROWS, DH), jnp.float32),
            pltpu.VMEM_SHARED((U, DH), jnp.float32),
            pltpu.SemaphoreType.DMA,
            pltpu.SemaphoreType.DMA,
            pltpu.SemaphoreType.DMA,
            pltpu.SemaphoreType.DMA,
            pltpu.SemaphoreType.DMA,
        ],
        compiler_params=pltpu.CompilerParams(needs_layout_passes=False),
    )
    return f(dst_u, dst_i, vals, ie_cat, ue_cat)


def _tc_layer_kernel(a_ref, w_ref, al_ref, s_ref, m_ref):
    al = al_ref[0, 0]
    w = w_ref[...]
    dot = functools.partial(jnp.dot, preferred_element_type=jnp.float32)
    z_prev = None
    for b in range(3):
        z = dot(a_ref[b], w)
        y = z if z_prev is None else z - z_prev
        s_ref[b, :, :] = jnp.where(y > 0, y, al * y)
        z_prev = z
    m = z_prev * (1.0 / 3.0)
    m = jnp.where(m > 0, m, al * m)
    m_ref[0, :, :] = m[:, :DH]
    m_ref[1, :, :] = m[:, DH:]


def _tc_layer(embs, w, alpha):
    return pl.pallas_call(
        _tc_layer_kernel,
        grid=(U // BM,),
        in_specs=[
            pl.BlockSpec((3, BM, D), lambda i: (0, i, 0)),
            pl.BlockSpec((D, D), lambda i: (0, 0)),
            pl.BlockSpec(memory_space=pltpu.SMEM),
        ],
        out_specs=[
            pl.BlockSpec((3, BM, D), lambda i: (0, i, 0)),
            pl.BlockSpec((2, BM, DH), lambda i: (0, i, 0)),
        ],
        out_shape=[
            jax.ShapeDtypeStruct((3, U, D), jnp.float32),
            jax.ShapeDtypeStruct((2, U, DH), jnp.float32),
        ],
    )(embs, w, alpha.reshape(1, 1))


def _tc_final_kernel(m0_ref, m1_ref, s0_ref, s1_ref, w_ref, emb_ref, embs_ref):
    w = w_ref[...]
    dot = functools.partial(jnp.dot, preferred_element_type=jnp.float32)
    emb_ref[...] = (dot(m0_ref[0], w[:DH])
                    + dot(m0_ref[1], w[DH:D])
                    + dot(m1_ref[0], w[D:D + DH])
                    + dot(m1_ref[1], w[D + DH:]))
    for b in range(3):
        embs_ref[b, :, :] = (dot(s0_ref[b], w[:D]) + dot(s1_ref[b], w[D:]))


def _tc_final(m0, m1, s0, s1, wcat):
    hspec = pl.BlockSpec((2, BM, DH), lambda i: (0, i, 0))
    sspec = pl.BlockSpec((3, BM, D), lambda i: (0, i, 0))
    return pl.pallas_call(
        _tc_final_kernel,
        grid=(U // BM,),
        in_specs=[
            hspec, hspec,
            sspec, sspec,
            pl.BlockSpec((2 * D, D), lambda i: (0, 0)),
        ],
        out_specs=[
            pl.BlockSpec((BM, D), lambda i: (i, 0)),
            sspec,
        ],
        out_shape=[
            jax.ShapeDtypeStruct((U, D), jnp.float32),
            jax.ShapeDtypeStruct((3, U, D), jnp.float32),
        ],
    )(m0, m1, s0, s1, wcat)


def kernel(user_emb, item_emb, rows0, cols0, vals0, rows1, cols1, vals1,
           rows2, cols2, vals2, u_w0, i_w0, alpha0, u_w1, i_w1, alpha1,
           u_concat_w, i_concat_w):
    pad_idx = jnp.arange(E_PAD - E, dtype=jnp.int32)
    pad_val = jnp.zeros((E_PAD - E,), jnp.float32)

    def prep_i(x):
        return jnp.concatenate([x.astype(jnp.int32), pad_idx]).reshape(
            ROWS_B, K)

    def prep_f(x):
        return jnp.concatenate([x, pad_val]).reshape(ROWS_B, K)

    dst_u = jnp.concatenate([prep_i(rows0), prep_i(rows1), prep_i(rows2)])
    dst_i = jnp.concatenate([prep_i(cols0), prep_i(cols1), prep_i(cols2)])
    vals = jnp.concatenate([prep_f(vals0), prep_f(vals1), prep_f(vals2)])

    def to_cat(x):
        return jnp.transpose(x.reshape(U, 2, DH), (1, 0, 2)).reshape(2 * U, DH)

    ue_cat = to_cat(user_emb)
    ie_cat = to_cat(item_emb)

    u_embs0, i_embs0 = _sc_layer(dst_u, dst_i, vals, ie_cat, ue_cat)
    s_u0, mu0 = _tc_layer(u_embs0, u_w0, alpha0)
    s_i0, mi0 = _tc_layer(i_embs0, i_w0, alpha0)

    u_embs1, i_embs1 = _sc_layer(dst_u, dst_i, vals,
                                 mi0.reshape(2 * U, DH), mu0.reshape(2 * U, DH))
    s_u1, mu1 = _tc_layer(u_embs1, u_w1, alpha1)
    s_i1, mi1 = _tc_layer(i_embs1, i_w1, alpha1)

    user_embedding, user_embeddings = _tc_final(mu0, mu1, s_u0, s_u1,
                                                u_concat_w)
    item_embedding, item_embeddings = _tc_final(mi0, mi1, s_i0, s_i1,
                                                i_concat_w)

    return (user_embedding, item_embedding, user_embeddings, item_embeddings)


# speedup vs baseline: 1.1822x; 1.0258x over previous
import functools

import jax
import jax.numpy as jnp
from jax import lax
from jax.experimental import pallas as pl
from jax.experimental.pallas import tpu as pltpu
from jax.experimental.pallas import tpu_sc as plsc

U = 10000
D = 256
DH = 128
E = 160000
NC = 2
NS = 16
LANES = 16
K = 80
E_PAD = 163840
ROWS_B = E_PAD // K
CHUNKS = E_PAD // NS // K
SCHUNKS = 16
NSTAGE = CHUNKS // SCHUNKS
DRAIN_TILES = 10
DRAIN_ROWS = U // DRAIN_TILES
ZROWS = 40
BM = 1000


def _sc_dir_body(dst_rc_a, src_rc_a, vals, x_cat, out_ref,
                 sidx_v, didx_v, vals_v, rows0_v, rows1_v, zero_v, acc_sh,
                 gsem0, gsem1, ssem0, ssem1, zsem):
    cid = lax.axis_index("c")
    sid = lax.axis_index("s")
    drow0 = sid * DRAIN_ROWS
    half_off = jnp.full((LANES,), cid * U, jnp.int32)

    zvec = jnp.zeros((LANES,), jnp.float32)

    @plsc.parallel_loop(0, ZROWS, 1, unroll=2)
    def _(i):
        for t in range(DH // LANES):
            zero_v[i, pl.ds(t * LANES, LANES)] = zvec

    rows_bufs = (rows0_v, rows1_v)
    gsems = (gsem0, gsem1)
    ssems = (ssem0, ssem1)

    def gather_start(x_cat, jj, b):
        pltpu.async_copy(x_cat.at[sidx_v.at[jj]], rows_bufs[b], gsems[b])

    def gather_wait(x_cat, jj, b):
        pltpu.make_async_copy(
            x_cat.at[sidx_v.at[jj]], rows_bufs[b], gsems[b]).wait()

    def scatter_start(jj, b):
        pltpu.async_copy(rows_bufs[b], acc_sh.at[didx_v.at[jj]], ssems[b],
                         add=True)

    def scatter_wait(b):
        pltpu.make_async_copy(
            rows_bufs[b], acc_sh.at[pl.ds(0, K)], ssems[b]).wait()

    def scale(jj, b):
        buf = rows_bufs[b]

        @plsc.parallel_loop(0, K, 1, unroll=2)
        def _(k):
            vk = plsc.load_gather(
                vals_v,
                [jnp.full((LANES,), jj, jnp.int32),
                 jnp.full((LANES,), k, jnp.int32)])
            for t in range(DH // LANES):
                buf[k, pl.ds(t * LANES, LANES)] = (
                    buf[k, pl.ds(t * LANES, LANES)] * vk)

    def stage_pass(dst_rc, src_rc, crow0):
        pltpu.sync_copy(src_rc.at[pl.ds(crow0, SCHUNKS)], sidx_v)
        pltpu.sync_copy(dst_rc.at[pl.ds(crow0, SCHUNKS)], didx_v)
        pltpu.sync_copy(vals.at[pl.ds(crow0, SCHUNKS)], vals_v)

        @plsc.parallel_loop(0, SCHUNKS, 1, unroll=2)
        def _(r):
            for t in range(K // LANES):
                sidx_v[r, pl.ds(t * LANES, LANES)] = (
                    sidx_v[r, pl.ds(t * LANES, LANES)] + half_off)

    def one_direction(dst_rc, src_rc, x_cat, out_ref):
        @pl.when(sid < DRAIN_TILES)
        def _():
            def zb(z, c):
                pltpu.async_copy(
                    zero_v, acc_sh.at[pl.ds(drow0 + z * ZROWS, ZROWS)], zsem)
                return c

            lax.fori_loop(0, DRAIN_ROWS // ZROWS, zb, 0)

            def zw(z, c):
                pltpu.make_async_copy(
                    zero_v, acc_sh.at[pl.ds(drow0, ZROWS)], zsem).wait()
                return c

            lax.fori_loop(0, DRAIN_ROWS // ZROWS, zw, 0)

        plsc.subcore_barrier()

        def behavior(b, carry):
            def stage(st, c2):
                crow0 = b * ROWS_B + sid * CHUNKS + st * SCHUNKS
                stage_pass(dst_rc, src_rc, crow0)

                gather_start(x_cat, 0, 0)

                def pair(j2, c3):
                    j0 = j2 * 2
                    gather_wait(x_cat, j0, 0)

                    @pl.when(j2 >= 1)
                    def _():
                        scatter_wait(1)

                    gather_start(x_cat, j0 + 1, 1)
                    scale(j0, 0)
                    scatter_start(j0, 0)
                    gather_wait(x_cat, j0 + 1, 1)

                    @pl.when(j2 + 1 < SCHUNKS // 2)
                    def _():
                        scatter_wait(0)
                        gather_start(x_cat, j0 + 2, 0)

                    scale(j0 + 1, 1)
                    scatter_start(j0 + 1, 1)
                    return c3

                lax.fori_loop(0, SCHUNKS // 2, pair, 0)
                scatter_wait(0)
                scatter_wait(1)
                return c2

            lax.fori_loop(0, NSTAGE, stage, 0)
            plsc.subcore_barrier()

            @pl.when(sid < DRAIN_TILES)
            def _():
                pltpu.sync_copy(
                    acc_sh.at[pl.ds(drow0, DRAIN_ROWS)],
                    out_ref.at[b, pl.ds(drow0, DRAIN_ROWS),
                               pl.ds(cid * DH, DH)])
            plsc.subcore_barrier()
            return carry

        lax.fori_loop(0, 3, behavior, 0)

    one_direction(dst_rc_a, src_rc_a, x_cat, out_ref)


def _sc_dir(dst_rc, src_rc, vals, x_cat):
    mesh = plsc.VectorSubcoreMesh(
        core_axis_name="c", subcore_axis_name="s",
        num_cores=NC, num_subcores=NS)
    f = pl.kernel(
        _sc_dir_body,
        out_type=jax.ShapeDtypeStruct((3, U, D), jnp.float32),
        mesh=mesh,
        scratch_types=[
            pltpu.VMEM((SCHUNKS, K), jnp.int32),
            pltpu.VMEM((SCHUNKS, K), jnp.int32),
            pltpu.VMEM((SCHUNKS, K), jnp.float32),
            pltpu.VMEM((K, DH), jnp.float32),
            pltpu.VMEM((K, DH), jnp.float32),
            pltpu.VMEM((ZROWS, DH), jnp.float32),
            pltpu.VMEM_SHARED((U, DH), jnp.float32),
            pltpu.SemaphoreType.DMA,
            pltpu.SemaphoreType.DMA,
            pltpu.SemaphoreType.DMA,
            pltpu.SemaphoreType.DMA,
            pltpu.SemaphoreType.DMA,
        ],
        compiler_params=pltpu.CompilerParams(needs_layout_passes=False),
    )
    return f(dst_rc, src_rc, vals, x_cat)


def _tc_layer_kernel(a_ref, w_ref, al_ref, s_ref, m_ref):
    al = al_ref[0, 0]
    w = w_ref[...]
    dot = functools.partial(jnp.dot, preferred_element_type=jnp.float32)
    z_prev = None
    for b in range(3):
        z = dot(a_ref[b], w)
        y = z if z_prev is None else z - z_prev
        s_ref[b, :, :] = jnp.where(y > 0, y, al * y)
        z_prev = z
    m = z_prev * (1.0 / 3.0)
    m = jnp.where(m > 0, m, al * m)
    m_ref[0, :, :] = m[:, :DH]
    m_ref[1, :, :] = m[:, DH:]


def _tc_layer(embs, w, alpha):
    return pl.pallas_call(
        _tc_layer_kernel,
        grid=(U // BM,),
        in_specs=[
            pl.BlockSpec((3, BM, D), lambda i: (0, i, 0)),
            pl.BlockSpec((D, D), lambda i: (0, 0)),
            pl.BlockSpec(memory_space=pltpu.SMEM),
        ],
        out_specs=[
            pl.BlockSpec((3, BM, D), lambda i: (0, i, 0)),
            pl.BlockSpec((2, BM, DH), lambda i: (0, i, 0)),
        ],
        out_shape=[
            jax.ShapeDtypeStruct((3, U, D), jnp.float32),
            jax.ShapeDtypeStruct((2, U, DH), jnp.float32),
        ],
    )(embs, w, alpha.reshape(1, 1))


def _tc_final_kernel(m0_ref, m1_ref, s0_ref, s1_ref, w_ref, emb_ref, embs_ref):
    w = w_ref[...]
    dot = functools.partial(jnp.dot, preferred_element_type=jnp.float32)
    emb_ref[...] = (dot(m0_ref[0], w[:DH])
                    + dot(m0_ref[1], w[DH:D])
                    + dot(m1_ref[0], w[D:D + DH])
                    + dot(m1_ref[1], w[D + DH:]))
    for b in range(3):
        embs_ref[b, :, :] = (dot(s0_ref[b], w[:D]) + dot(s1_ref[b], w[D:]))


def _tc_final(m0, m1, s0, s1, wcat):
    hspec = pl.BlockSpec((2, BM, DH), lambda i: (0, i, 0))
    sspec = pl.BlockSpec((3, BM, D), lambda i: (0, i, 0))
    return pl.pallas_call(
        _tc_final_kernel,
        grid=(U // BM,),
        in_specs=[
            hspec, hspec,
            sspec, sspec,
            pl.BlockSpec((2 * D, D), lambda i: (0, 0)),
        ],
        out_specs=[
            pl.BlockSpec((BM, D), lambda i: (i, 0)),
            sspec,
        ],
        out_shape=[
            jax.ShapeDtypeStruct((U, D), jnp.float32),
            jax.ShapeDtypeStruct((3, U, D), jnp.float32),
        ],
    )(m0, m1, s0, s1, wcat)


def kernel(user_emb, item_emb, rows0, cols0, vals0, rows1, cols1, vals1,
           rows2, cols2, vals2, u_w0, i_w0, alpha0, u_w1, i_w1, alpha1,
           u_concat_w, i_concat_w):
    pad_idx = jnp.arange(E_PAD - E, dtype=jnp.int32)
    pad_val = jnp.zeros((E_PAD - E,), jnp.float32)

    def prep_i(x):
        return jnp.concatenate([x.astype(jnp.int32), pad_idx]).reshape(
            ROWS_B, K)

    def prep_f(x):
        return jnp.concatenate([x, pad_val]).reshape(ROWS_B, K)

    dst_u = jnp.concatenate([prep_i(rows0), prep_i(rows1), prep_i(rows2)])
    dst_i = jnp.concatenate([prep_i(cols0), prep_i(cols1), prep_i(cols2)])
    vals = jnp.concatenate([prep_f(vals0), prep_f(vals1), prep_f(vals2)])

    def to_cat(x):
        return jnp.transpose(x.reshape(U, 2, DH), (1, 0, 2)).reshape(2 * U, DH)

    ue_cat = to_cat(user_emb)
    ie_cat = to_cat(item_emb)

    u_embs0 = _sc_dir(dst_u, dst_i, vals, ie_cat)
    i_embs0 = _sc_dir(dst_i, dst_u, vals, ue_cat)
    s_u0, mu0 = _tc_layer(u_embs0, u_w0, alpha0)
    s_i0, mi0 = _tc_layer(i_embs0, i_w0, alpha0)

    u_embs1 = _sc_dir(dst_u, dst_i, vals, mi0.reshape(2 * U, DH))
    i_embs1 = _sc_dir(dst_i, dst_u, vals, mu0.reshape(2 * U, DH))
    s_u1, mu1 = _tc_layer(u_embs1, u_w1, alpha1)
    s_i1, mi1 = _tc_layer(i_embs1, i_w1, alpha1)

    user_embedding, user_embeddings = _tc_final(mu0, mu1, s_u0, s_u1,
                                                u_concat_w)
    item_embedding, item_embeddings = _tc_final(mi0, mi1, s_i0, s_i1,
                                                i_concat_w)

    return (user_embedding, item_embedding, user_embeddings, item_embeddings)
